# Initial kernel scaffold; baseline (speedup 1.0000x reference)
import jax
import jax.numpy as jnp
from jax.experimental import pallas as pl


def kernel(x, edge_index, batch, edge_attr, Wr1, br1, Wo1, Wr2, br2, Wo2, Ws, bs, Wt, bt, W1, b1, W2, b2, Wf, bf):
    raise NotImplementedError("write your pallas kernel here")


# speedup vs baseline: 2.8228x; 2.8228x over previous
import functools

import jax
import jax.numpy as jnp
from jax import lax
from jax.experimental import pallas as pl
from jax.experimental.pallas import tpu as pltpu
from jax.experimental.pallas import tpu_sc as plsc

N = 10000
D = 128
G = 128
RELU_COEF = 0.05

NTILES = 32
CHUNKS = 80
CB = 128
EP = NTILES * CHUNKS * CB
NP = 10240
ROWS_PER_TILE = NP // 16


def _leaky(v):
    return jnp.where(v > 0, v, RELU_COEF * v)


def _dotT(a, w):
    return lax.dot_general(a, w, (((1,), (1,)), ((), ())),
                           preferred_element_type=jnp.float32)


def _sc_agg_body(x_hbm, src_hbm, dst_hbm, ew_hbm, out_hbm,
                 src_v, dst_v, ew_v, rows_v, acc, sem):
    cid = lax.axis_index("c")
    sid = lax.axis_index("s")
    wid = cid * 16 + sid

    pltpu.sync_copy(src_hbm.at[wid], src_v)
    pltpu.sync_copy(dst_hbm.at[wid], dst_v)
    pltpu.sync_copy(ew_hbm.at[wid], ew_v)

    def zrow(r, carry):
        for k in range(D // 16):
            rows_v[r, pl.ds(k * 16, 16)] = jnp.zeros((16,), jnp.float32)
        return carry
    lax.fori_loop(0, CB, zrow, 0)
    base = sid * ROWS_PER_TILE
    for j in range(ROWS_PER_TILE // CB):
        pltpu.sync_copy(rows_v, acc.at[pl.ds(base + j * CB, CB)])
    plsc.subcore_barrier()

    def chunk_body(c, carry):
        pltpu.async_copy(x_hbm.at[src_v.at[c]], rows_v, sem).wait()

        def edge_body(e, inner):
            w = plsc.load_gather(
                ew_v, [jnp.full((16,), c, jnp.int32),
                       jnp.full((16,), e, jnp.int32)])
            for k in range(D // 16):
                rows_v[e, pl.ds(k * 16, 16)] = (
                    rows_v[e, pl.ds(k * 16, 16)] * w)
            return inner
        lax.fori_loop(0, CB, edge_body, 0)

        pltpu.sync_copy(rows_v, acc.at[dst_v.at[c]], add=True)
        return carry
    lax.fori_loop(0, CHUNKS, chunk_body, 0)
    plsc.subcore_barrier()

    for j in range(ROWS_PER_TILE // CB):
        pltpu.sync_copy(acc.at[pl.ds(base + j * CB, CB)],
                        out_hbm.at[cid, pl.ds(base + j * CB, CB)])


_sc_agg = functools.partial(
    pl.kernel,
    out_type=jax.ShapeDtypeStruct((2, NP, D), jnp.float32),
    mesh=plsc.VectorSubcoreMesh(core_axis_name="c", subcore_axis_name="s"),
    scratch_types=[
        pltpu.VMEM((CHUNKS, CB), jnp.int32),
        pltpu.VMEM((CHUNKS, CB), jnp.int32),
        pltpu.VMEM((CHUNKS, CB), jnp.float32),
        pltpu.VMEM((CB, D), jnp.float32),
        pltpu.VMEM_SHARED((NP, D), jnp.float32),
        pltpu.SemaphoreType.DMA,
    ],
    compiler_params=pltpu.CompilerParams(needs_layout_passes=False),
)(_sc_agg_body)


BN = 1000
NGRID = N // BN


def _tc1_body(p_ref, x_ref, wr_ref, br_ref, wo_ref, o_ref):
    agg = p_ref[0] + p_ref[1]
    h = _dotT(agg, wr_ref[...]) + br_ref[...] + _dotT(x_ref[...], wo_ref[...])
    o_ref[...] = _leaky(h)


def _tc_dense1(p, x, Wr1, br1, Wo1):
    return pl.pallas_call(
        _tc1_body,
        grid=(NGRID,),
        in_specs=[
            pl.BlockSpec((2, BN, D), lambda i: (0, i, 0)),
            pl.BlockSpec((BN, D), lambda i: (i, 0)),
            pl.BlockSpec((D, D), lambda i: (0, 0)),
            pl.BlockSpec((1, D), lambda i: (0, 0)),
            pl.BlockSpec((D, D), lambda i: (0, 0)),
        ],
        out_specs=pl.BlockSpec((BN, D), lambda i: (i, 0)),
        out_shape=jax.ShapeDtypeStruct((N, D), jnp.float32),
    )(p, x, Wr1, br1, Wo1)


def _tc2_body(q_ref, h1_ref, x_ref, b_ref,
              wr_ref, br_ref, wo_ref,
              wsh_ref, wsx_ref, bs_ref, wth_ref, wtx_ref, bt_ref,
              w1_ref, b1_ref, w2_ref, b2_ref, wf_ref, bf_ref,
              o_ref, acc_ref):
    i = pl.program_id(0)
    agg = q_ref[0] + q_ref[1]
    h1 = h1_ref[...]
    x = x_ref[...]
    h2 = _leaky(_dotT(agg, wr_ref[...]) + br_ref[...]
                + _dotT(h1, wo_ref[...]))
    sx = _dotT(h2, wsh_ref[...]) + _dotT(x, wsx_ref[...]) + bs_ref[...]
    tx = _dotT(h2, wth_ref[...]) + _dotT(x, wtx_ref[...]) + bt_ref[...]
    sx = jnp.clip(sx, -30.0, 30.0)
    h3 = (1.0 / (1.0 + jnp.exp(sx))) * jnp.tanh(tx)

    g_iota = lax.broadcasted_iota(jnp.int32, (G, BN), 0)
    oh = (b_ref[0] == g_iota).astype(jnp.float32)
    pooled = lax.dot_general(oh, h3, (((1,), (0,)), ((), ())),
                             preferred_element_type=jnp.float32)

    @pl.when(i == 0)
    def _():
        acc_ref[...] = pooled

    @pl.when(i > 0)
    def _():
        acc_ref[...] = acc_ref[...] + pooled

    @pl.when(i == NGRID - 1)
    def _():
        f = _leaky(_dotT(acc_ref[...], w1_ref[...]) + b1_ref[...])
        f = _leaky(_dotT(f, w2_ref[...]) + b2_ref[...])
        out = jnp.sum(f * wf_ref[...], axis=1, keepdims=True) + bf_ref[0, 0]
        o_ref[...] = 1.0 / (1.0 + jnp.exp(-out))


def _tc_head(q, h1, x, batch2d, Wr2, br2, Wo2,
             Wsh, Wsx, bs, Wth, Wtx, bt, W1, b1, W2, b2, Wf, bf):
    full = lambda shape: pl.BlockSpec(shape, lambda i: tuple(0 for _ in shape))
    return pl.pallas_call(
        _tc2_body,
        grid=(NGRID,),
        in_specs=[
            pl.BlockSpec((2, BN, D), lambda i: (0, i, 0)),
            pl.BlockSpec((BN, D), lambda i: (i, 0)),
            pl.BlockSpec((BN, D), lambda i: (i, 0)),
            pl.BlockSpec((1, 1, BN), lambda i: (i, 0, 0)),
            full((D, D)), full((1, D)), full((D, D)),
            full((D, D)), full((D, D)), full((1, D)),
            full((D, D)), full((D, D)), full((1, D)),
            full((D, D)), full((1, D)),
            full((258, D)), full((1, 258)),
            full((1, 258)), full((1, 1)),
        ],
        out_specs=pl.BlockSpec((G, 1), lambda i: (0, 0)),
        out_shape=jax.ShapeDtypeStruct((G, 1), jnp.float32),
        scratch_shapes=[pltpu.VMEM((G, D), jnp.float32)],
    )(q, h1, x, batch2d, Wr2, br2, Wo2,
      Wsh, Wsx, bs, Wth, Wtx, bt, W1, b1, W2, b2, Wf, bf)


def kernel(x, edge_index, batch, edge_attr, Wr1, br1, Wo1, Wr2, br2, Wo2,
           Ws, bs, Wt, bt, W1, b1, W2, b2, Wf, bf):
    E = edge_index.shape[1]
    pad = EP - E
    src3 = jnp.pad(edge_index[0], (0, pad)).reshape(NTILES, CHUNKS, CB)
    dst3 = jnp.pad(edge_index[1], (0, pad)).reshape(NTILES, CHUNKS, CB)
    ew3 = jnp.pad(edge_attr, (0, pad)).reshape(NTILES, CHUNKS, CB)
    batch2d = batch.reshape(NGRID, 1, BN)

    br1_2 = br1.reshape(1, D)
    br2_2 = br2.reshape(1, D)
    bs_2 = bs.reshape(1, D)
    bt_2 = bt.reshape(1, D)
    b1_2 = b1.reshape(1, D)
    b2_2 = b2.reshape(1, 258)
    bf_2 = bf.reshape(1, 1)
    Wsh, Wsx = Ws[:, :D], Ws[:, D:]
    Wth, Wtx = Wt[:, :D], Wt[:, D:]

    p = _sc_agg(x, src3, dst3, ew3)
    h1 = _tc_dense1(p, x, Wr1, br1_2, Wo1)
    q = _sc_agg(h1, src3, dst3, ew3)
    return _tc_head(q, h1, x, batch2d, Wr2, br2_2, Wo2,
                    Wsh, Wsx, bs_2, Wth, Wtx, bt_2,
                    W1, b1_2, W2, b2_2, Wf, bf_2)
